# baseline (speedup 1.0000x reference)
import functools

import jax
import jax.numpy as jnp
from jax.experimental import pallas as pl
from jax.experimental.pallas import tpu as pltpu

EPS = 1e-05
CHUNK = 2048
SUB = 512


def _body(x_ref, m_ref, p_ref, wb_ref,
          y_ref, o_ref,
          s_acc, q_acc, c_acc, *, nc, d):
    c = pl.program_id(1)

    @pl.when(c == 0)
    def _init():
        c0 = p_ref[0, 0:1, 2 * d:2 * d + 128]
        mu0 = p_ref[0, 0:1, 0:d]
        v0 = p_ref[0, 0:1, d:2 * d]
        c_acc[...] = c0
        s_acc[...] = c0[:, 0:1] * mu0
        q_acc[...] = c0[:, 0:1] * (v0 + mu0 * mu0)

    chunk = x_ref.shape[1]
    sub = SUB

    row = jax.lax.broadcasted_iota(jnp.int32, (sub, sub), 0)
    col = jax.lax.broadcasted_iota(jnp.int32, (sub, sub), 1)
    lower = col <= row
    ones128 = jnp.ones((sub, 128), jnp.bfloat16)
    gamma = wb_ref[0, 0:1, 0:d] + 1.0
    beta = wb_ref[0, 0:1, d:2 * d]

    c_prev = c_acc[0:1, 0:1]
    s_prev = s_acc[0:1, :]
    q_prev = q_acc[0:1, :]

    mean = s_prev
    var = q_prev
    cnt_last = c_prev

    for g in range(chunk // sub):
        x = x_ref[0, g * sub:(g + 1) * sub, :]
        m_row = m_ref[0, 0:1, pl.ds(c * chunk + g * sub, sub)]

        m_b = jnp.broadcast_to(m_row, (sub, sub))
        trim = jnp.where(lower, m_b, 0.0).astype(jnp.bfloat16)

        cum_m = jax.lax.dot(trim, ones128,
                            preferred_element_type=jnp.float32)[:, 0:1]
        m_col = cum_m - jnp.concatenate(
            [jnp.zeros((1, 1), jnp.float32), cum_m[:sub - 1, :]], axis=0)

        cum_x = jax.lax.dot(trim, x.astype(jnp.bfloat16),
                            preferred_element_type=jnp.float32)
        cum_x2 = jax.lax.dot(trim, (x * x).astype(jnp.bfloat16),
                             preferred_element_type=jnp.float32)

        cnt = c_prev + cum_m
        s = s_prev + cum_x
        q = q_prev + cum_x2

        inv = 1.0 / cnt
        mean = s * inv
        var = q * inv - mean * mean
        y = (gamma * (x - mean) * jax.lax.rsqrt(var + EPS) + beta) * m_col
        y_ref[0, g * sub:(g + 1) * sub, :] = y

        c_prev = cnt[sub - 1:sub, :]
        s_prev = s[sub - 1:sub, :]
        q_prev = q[sub - 1:sub, :]
        cnt_last = c_prev

    s_acc[...] = s_prev
    q_acc[...] = q_prev
    c_acc[...] = jnp.broadcast_to(cnt_last, (1, 128))

    @pl.when(c == nc - 1)
    def _final():
        o_ref[0, 0:1, 0:d] = mean[sub - 1:sub, :]
        o_ref[0, 0:1, d:2 * d] = jnp.maximum(var[sub - 1:sub, :], 0.0)
        o_ref[0, 0:1, 2 * d:2 * d + 128] = jnp.broadcast_to(cnt_last, (1, 128))


def kernel(x, prev_count, prev_mean, prev_var, weight, bias, padding_mask):
    B, L, D = x.shape
    cl = CHUNK
    nc = L // cl
    valid = (~padding_mask).astype(jnp.float32).reshape(B, 1, L)
    c0b = jnp.broadcast_to(prev_count.astype(jnp.float32)[:, None, None],
                           (B, 1, 128))
    priors = jnp.concatenate(
        [prev_mean.reshape(B, 1, D), prev_var.reshape(B, 1, D), c0b], axis=2)
    wb = jnp.concatenate(
        [weight.reshape(1, 1, D), bias.reshape(1, 1, D)], axis=2)

    grid = (B, nc)
    kern = pl.pallas_call(
        functools.partial(_body, nc=nc, d=D),
        grid=grid,
        in_specs=[
            pl.BlockSpec((1, cl, D), lambda b, c: (b, c, 0)),
            pl.BlockSpec((1, 1, L), lambda b, c: (b, 0, 0)),
            pl.BlockSpec((1, 1, 2 * D + 128), lambda b, c: (b, 0, 0)),
            pl.BlockSpec((1, 1, 2 * D), lambda b, c: (0, 0, 0)),
        ],
        out_specs=[
            pl.BlockSpec((1, cl, D), lambda b, c: (b, c, 0)),
            pl.BlockSpec((1, 1, 2 * D + 128), lambda b, c: (b, 0, 0)),
        ],
        out_shape=[
            jax.ShapeDtypeStruct((B, L, D), jnp.float32),
            jax.ShapeDtypeStruct((B, 1, 2 * D + 128), jnp.float32),
        ],
        scratch_shapes=[
            pltpu.VMEM((1, D), jnp.float32),
            pltpu.VMEM((1, D), jnp.float32),
            pltpu.VMEM((1, 128), jnp.float32),
        ],
        compiler_params=pltpu.CompilerParams(
            dimension_semantics=("parallel", "arbitrary"),
        ),
    )
    y, out = kern(x, valid, priors, wb)
    return (y, out[:, 0, 2 * D], out[:, 0, 0:D], out[:, 0, D:2 * D])


# speedup vs baseline: 1.1996x; 1.1996x over previous
import functools

import jax
import jax.numpy as jnp
from jax.experimental import pallas as pl
from jax.experimental.pallas import tpu as pltpu

EPS = 1e-05
CHUNK = 2048
SUB = 256


def _body(x_ref, m_ref, p_ref, wb_ref,
          y_ref, o_ref,
          s_acc, q_acc, c_acc, *, nc, d):
    c = pl.program_id(1)

    @pl.when(c == 0)
    def _init():
        c0 = p_ref[0, 0:1, 2 * d:2 * d + 128]
        mu0 = p_ref[0, 0:1, 0:d]
        v0 = p_ref[0, 0:1, d:2 * d]
        c_acc[...] = c0
        s_acc[...] = c0[:, 0:1] * mu0
        q_acc[...] = c0[:, 0:1] * (v0 + mu0 * mu0)

    chunk = x_ref.shape[1]
    sub = SUB

    row = jax.lax.broadcasted_iota(jnp.int32, (sub, sub), 0)
    col = jax.lax.broadcasted_iota(jnp.int32, (sub, sub), 1)
    lower = col <= row
    ones128 = jnp.ones((sub, 128), jnp.bfloat16)
    gamma = wb_ref[0, 0:1, 0:d] + 1.0
    beta = wb_ref[0, 0:1, d:2 * d]

    c_prev = c_acc[0:1, 0:1]
    s_prev = s_acc[0:1, :]
    q_prev = q_acc[0:1, :]

    mean = s_prev
    var = q_prev
    cnt_last = c_prev

    for g in range(chunk // sub):
        x = x_ref[0, g * sub:(g + 1) * sub, :]
        m_row = m_ref[0, 0:1, pl.ds(c * chunk + g * sub, sub)]

        m_b = jnp.broadcast_to(m_row, (sub, sub))
        trim = jnp.where(lower, m_b, 0.0).astype(jnp.bfloat16)

        cum_m = jax.lax.dot(trim, ones128,
                            preferred_element_type=jnp.float32)[:, 0:1]
        m_col = cum_m - jnp.concatenate(
            [jnp.zeros((1, 1), jnp.float32), cum_m[:sub - 1, :]], axis=0)

        x_bf = x.astype(jnp.bfloat16)
        cum_x = jax.lax.dot(trim, x_bf,
                            preferred_element_type=jnp.float32)
        cum_x2 = jax.lax.dot(trim, x_bf * x_bf,
                             preferred_element_type=jnp.float32)

        cnt = c_prev + cum_m
        s = s_prev + cum_x
        q = q_prev + cum_x2

        inv = 1.0 / cnt
        mean = s * inv
        var = q * inv - mean * mean
        y = (gamma * (x - mean) * jax.lax.rsqrt(var + EPS) + beta) * m_col
        y_ref[0, g * sub:(g + 1) * sub, :] = y

        c_prev = cnt[sub - 1:sub, :]
        s_prev = s[sub - 1:sub, :]
        q_prev = q[sub - 1:sub, :]
        cnt_last = c_prev

    s_acc[...] = s_prev
    q_acc[...] = q_prev
    c_acc[...] = jnp.broadcast_to(cnt_last, (1, 128))

    @pl.when(c == nc - 1)
    def _final():
        o_ref[0, 0:1, 0:d] = mean[sub - 1:sub, :]
        o_ref[0, 0:1, d:2 * d] = jnp.maximum(var[sub - 1:sub, :], 0.0)
        o_ref[0, 0:1, 2 * d:2 * d + 128] = jnp.broadcast_to(cnt_last, (1, 128))


def kernel(x, prev_count, prev_mean, prev_var, weight, bias, padding_mask):
    B, L, D = x.shape
    cl = CHUNK
    nc = L // cl
    valid = (~padding_mask).astype(jnp.float32).reshape(B, 1, L)
    c0b = jnp.broadcast_to(prev_count.astype(jnp.float32)[:, None, None],
                           (B, 1, 128))
    priors = jnp.concatenate(
        [prev_mean.reshape(B, 1, D), prev_var.reshape(B, 1, D), c0b], axis=2)
    wb = jnp.concatenate(
        [weight.reshape(1, 1, D), bias.reshape(1, 1, D)], axis=2)

    grid = (B, nc)
    kern = pl.pallas_call(
        functools.partial(_body, nc=nc, d=D),
        grid=grid,
        in_specs=[
            pl.BlockSpec((1, cl, D), lambda b, c: (b, c, 0)),
            pl.BlockSpec((1, 1, L), lambda b, c: (b, 0, 0)),
            pl.BlockSpec((1, 1, 2 * D + 128), lambda b, c: (b, 0, 0)),
            pl.BlockSpec((1, 1, 2 * D), lambda b, c: (0, 0, 0)),
        ],
        out_specs=[
            pl.BlockSpec((1, cl, D), lambda b, c: (b, c, 0)),
            pl.BlockSpec((1, 1, 2 * D + 128), lambda b, c: (b, 0, 0)),
        ],
        out_shape=[
            jax.ShapeDtypeStruct((B, L, D), jnp.float32),
            jax.ShapeDtypeStruct((B, 1, 2 * D + 128), jnp.float32),
        ],
        scratch_shapes=[
            pltpu.VMEM((1, D), jnp.float32),
            pltpu.VMEM((1, D), jnp.float32),
            pltpu.VMEM((1, 128), jnp.float32),
        ],
        compiler_params=pltpu.CompilerParams(
            dimension_semantics=("parallel", "arbitrary"),
        ),
    )
    y, out = kern(x, valid, priors, wb)
    return (y, out[:, 0, 2 * D], out[:, 0, 0:D], out[:, 0, D:2 * D])
